# Initial kernel scaffold; baseline (speedup 1.0000x reference)
import jax
import jax.numpy as jnp
from jax.experimental import pallas as pl


def kernel(X, target):
    raise NotImplementedError("write your pallas kernel here")


# speedup vs baseline: 1.1837x; 1.1837x over previous
import functools

import jax
import jax.numpy as jnp
from jax import lax
from jax.experimental import pallas as pl


_BN = 8


def _loss_body(x_ref, tgt_ref, out_ref):
    x = x_ref[...]
    bn, v = x.shape
    m = jnp.max(x, axis=-1, keepdims=True) * 0.5
    lo = m - 1.0
    hi = m
    for _ in range(9):
        width = hi - lo
        count = jnp.zeros_like(lo)
        for j in (1.0, 2.0, 3.0):
            tau = lo + width * (j * 0.25)
            t = jnp.maximum(x * 0.5 - tau, 0.0)
            s = jnp.sum(t * t, axis=-1, keepdims=True)
            count = count + (s >= 1.0).astype(jnp.float32)
        new_lo = lo + width * (count * 0.25)
        new_hi = lo + width * ((count + 1.0) * 0.25)
        lo, hi = new_lo, new_hi
    tau = 0.5 * (lo + hi)
    t = jnp.maximum(x * 0.5 - tau, 0.0)
    t2 = t * t
    s = jnp.sum(t2, axis=-1, keepdims=True)
    sp = jnp.sum(t2 * t, axis=-1, keepdims=True)
    dpx = jnp.sum(t2 * x, axis=-1, keepdims=True)
    tgt = tgt_ref[...]
    col = lax.broadcasted_iota(jnp.int32, (bn, v), 1)
    xt = jnp.sum(jnp.where(col == tgt, x, 0.0), axis=-1, keepdims=True)
    omega = (1.0 - sp / (s * jnp.sqrt(s))) / 0.75
    out_ref[...] = omega + dpx / s - xt


def _row_losses(X, target2d):
    n, v = X.shape
    grid = n // _BN
    return pl.pallas_call(
        _loss_body,
        grid=(grid,),
        in_specs=[
            pl.BlockSpec((_BN, v), lambda i: (i, 0)),
            pl.BlockSpec((_BN, 1), lambda i: (i, 0)),
        ],
        out_specs=pl.BlockSpec((_BN, 1), lambda i: (i, 0)),
        out_shape=jax.ShapeDtypeStruct((n, 1), jnp.float32),
    )(X, target2d)


@jax.jit
def kernel(X, target):
    n = X.shape[0]
    losses = _row_losses(X, target.reshape(n, 1))
    return jnp.sum(losses) / float(n)


# speedup vs baseline: 1.9594x; 1.6553x over previous
import functools

import jax
import jax.numpy as jnp
from jax import lax
from jax.experimental import pallas as pl


_BN = 8


def _loss_body(x_ref, tgt_ref, out_ref):
    x = x_ref[...]
    bn, v = x.shape
    m = jnp.max(x, axis=-1, keepdims=True) * 0.5
    tau = m - 1.0
    for _ in range(7):
        t = jnp.maximum(x * 0.5 - tau, 0.0)
        s1 = jnp.sum(t, axis=-1, keepdims=True)
        s2 = jnp.sum(t * t, axis=-1, keepdims=True)
        tau = tau + (s2 - 1.0) / (2.0 * s1 + 1e-30)
    t = jnp.maximum(x * 0.5 - tau, 0.0)
    t2 = t * t
    s = jnp.sum(t2, axis=-1, keepdims=True)
    sp = jnp.sum(t2 * t, axis=-1, keepdims=True)
    dpx = jnp.sum(t2 * x, axis=-1, keepdims=True)
    tgt = tgt_ref[...]
    col = lax.broadcasted_iota(jnp.int32, (bn, v), 1)
    xt = jnp.sum(jnp.where(col == tgt, x, 0.0), axis=-1, keepdims=True)
    omega = (1.0 - sp / (s * jnp.sqrt(s))) / 0.75
    out_ref[...] = omega + dpx / s - xt


def _row_losses(X, target2d):
    n, v = X.shape
    grid = n // _BN
    return pl.pallas_call(
        _loss_body,
        grid=(grid,),
        in_specs=[
            pl.BlockSpec((_BN, v), lambda i: (i, 0)),
            pl.BlockSpec((_BN, 1), lambda i: (i, 0)),
        ],
        out_specs=pl.BlockSpec((_BN, 1), lambda i: (i, 0)),
        out_shape=jax.ShapeDtypeStruct((n, 1), jnp.float32),
    )(X, target2d)


@jax.jit
def kernel(X, target):
    n = X.shape[0]
    losses = _row_losses(X, target.reshape(n, 1))
    return jnp.sum(losses) / float(n)


# speedup vs baseline: 2.8045x; 1.4313x over previous
import functools

import jax
import jax.numpy as jnp
from jax import lax
from jax.experimental import pallas as pl
from jax.experimental.pallas import tpu as pltpu


_BN = 8
_C = 1024


def _chunks(v):
    n_full = v // _C
    bounds = [(c * _C, _C) for c in range(n_full)]
    if v % _C:
        bounds.append((n_full * _C, v % _C))
    return bounds


def _loss_body(tgt_ref, x_ref, out_ref):
    v = x_ref.shape[1]
    bounds = _chunks(v)

    parts = []
    for (st, w) in bounds:
        xs = x_ref[:, st:st + w]
        parts.append(jnp.max(xs, axis=-1, keepdims=True))
    m = functools.reduce(jnp.maximum, parts) * 0.5

    tau = m - 1.0
    for _ in range(7):
        s1_parts, s2_parts = [], []
        acc1 = jnp.zeros((_BN, _C), jnp.float32)
        acc2 = jnp.zeros((_BN, _C), jnp.float32)
        for (st, w) in bounds:
            xs = x_ref[:, st:st + w]
            t = jnp.maximum(xs * 0.5 - tau, 0.0)
            if w == _C:
                acc1 = acc1 + t
                acc2 = acc2 + t * t
            else:
                s1_parts.append(jnp.sum(t, axis=-1, keepdims=True))
                s2_parts.append(jnp.sum(t * t, axis=-1, keepdims=True))
        s1_parts.append(jnp.sum(acc1, axis=-1, keepdims=True))
        s2_parts.append(jnp.sum(acc2, axis=-1, keepdims=True))
        s1 = sum(s1_parts)
        s2 = sum(s2_parts)
        tau = tau + (s2 - 1.0) / (2.0 * s1 + 1e-30)

    tgt = tgt_ref[...]
    base_col = lax.broadcasted_iota(jnp.int32, (_BN, _C), 1)
    acc_s = jnp.zeros((_BN, _C), jnp.float32)
    acc_sp = jnp.zeros((_BN, _C), jnp.float32)
    acc_dpx = jnp.zeros((_BN, _C), jnp.float32)
    acc_xt = jnp.zeros((_BN, _C), jnp.float32)
    s_parts, sp_parts, dpx_parts, xt_parts = [], [], [], []
    for (st, w) in bounds:
        xs = x_ref[:, st:st + w]
        t = jnp.maximum(xs * 0.5 - tau, 0.0)
        t2 = t * t
        if w == _C:
            hit = jnp.where(base_col == tgt - st, xs, 0.0)
            acc_s = acc_s + t2
            acc_sp = acc_sp + t2 * t
            acc_dpx = acc_dpx + t2 * xs
            acc_xt = acc_xt + hit
        else:
            col = lax.broadcasted_iota(jnp.int32, (_BN, w), 1)
            hit = jnp.where(col == tgt - st, xs, 0.0)
            s_parts.append(jnp.sum(t2, axis=-1, keepdims=True))
            sp_parts.append(jnp.sum(t2 * t, axis=-1, keepdims=True))
            dpx_parts.append(jnp.sum(t2 * xs, axis=-1, keepdims=True))
            xt_parts.append(jnp.sum(hit, axis=-1, keepdims=True))
    s_parts.append(jnp.sum(acc_s, axis=-1, keepdims=True))
    sp_parts.append(jnp.sum(acc_sp, axis=-1, keepdims=True))
    dpx_parts.append(jnp.sum(acc_dpx, axis=-1, keepdims=True))
    xt_parts.append(jnp.sum(acc_xt, axis=-1, keepdims=True))
    s = sum(s_parts)
    sp = sum(sp_parts)
    dpx = sum(dpx_parts)
    xt = sum(xt_parts)

    omega = (1.0 - sp / (s * jnp.sqrt(s))) / 0.75
    out_ref[...] = omega + dpx / s - xt


def _row_losses(X, target2d):
    n, v = X.shape
    grid = n // _BN
    return pl.pallas_call(
        _loss_body,
        grid=(grid,),
        in_specs=[
            pl.BlockSpec((_BN, 1), lambda i: (i, 0)),
            pl.BlockSpec((_BN, v), lambda i: (i, 0)),
        ],
        out_specs=pl.BlockSpec((_BN, 1), lambda i: (i, 0)),
        out_shape=jax.ShapeDtypeStruct((n, 1), jnp.float32),
    )(target2d, X)


@jax.jit
def kernel(X, target):
    n = X.shape[0]
    losses = _row_losses(X, target.reshape(n, 1))
    return jnp.sum(losses) / float(n)
